# baseline (speedup 1.0000x reference)
import functools

import jax
import jax.numpy as jnp
from jax import lax
from jax.experimental import pallas as pl
from jax.experimental.pallas import tpu as pltpu
from jax.experimental.pallas import tpu_sc as plsc

HIDDEN = 2048
NUM_CORES = 2
NUM_SUBCORES = 16
NUM_WORKERS = NUM_CORES * NUM_SUBCORES
CHUNK = 8
NBUF = 4
NSLOT = 2


def _emb_body(table_hbm, ids_hbm, out_hbm, idx_v, spmem, *rest):
    bufs = rest[:NBUF]
    gsems = rest[NBUF:2 * NBUF]
    ssems = rest[2 * NBUF:3 * NBUF]
    osems = rest[3 * NBUF:3 * NBUF + NSLOT]
    b_per_w = idx_v.shape[0]
    nchunk = b_per_w // CHUNK
    ngroup = nchunk // NBUF
    sid = lax.axis_index("s")
    wid = sid * NUM_CORES + lax.axis_index("c")
    base = wid * b_per_w
    pltpu.sync_copy(ids_hbm.at[pl.ds(base, b_per_w)], idx_v)

    def slot(s):
        return pl.ds((sid * NSLOT + s) * CHUNK, CHUNK)

    def g_start(c, b):
        pltpu.async_copy(
            table_hbm.at[idx_v.at[pl.ds(c * CHUNK, CHUNK)]], bufs[b], gsems[b]
        )

    def g_wait(b):
        pltpu.make_async_copy(table_hbm.at[pl.ds(0, CHUNK)], bufs[b],
                              gsems[b]).wait()

    def s_start(b, s):
        pltpu.async_copy(bufs[b], spmem.at[slot(s)], ssems[b])

    def s_wait(b, s):
        pltpu.make_async_copy(bufs[b], spmem.at[slot(s)], ssems[b]).wait()

    def o_start(c, s):
        pltpu.async_copy(spmem.at[slot(s)],
                         out_hbm.at[pl.ds(base + c * CHUNK, CHUNK)], osems[s])

    def o_wait(s):
        pltpu.make_async_copy(spmem.at[slot(s)],
                              out_hbm.at[pl.ds(base, CHUNK)], osems[s]).wait()

    def group_step(p, carry):
        for b in range(NBUF):
            g_start(p * NBUF + b, b)
        for b in range(NBUF):
            c = p * NBUF + b
            s = b % NSLOT
            g_wait(b)

            @pl.when((p > 0) | (b >= NSLOT))
            def _():
                o_wait(s)

            s_start(b, s)
            s_wait(b, s)
            o_start(c, s)
        return carry

    lax.fori_loop(0, ngroup, group_step, 0)
    for s in range(NSLOT):
        o_wait(s)


def kernel(input_ids, word_embeddings):
    batch, seq = input_ids.shape
    total = batch * seq
    b_per_w = total // NUM_WORKERS
    ids = input_ids.reshape(total).astype(jnp.int32)

    mesh = plsc.VectorSubcoreMesh(core_axis_name="c", subcore_axis_name="s")
    out = pl.kernel(
        _emb_body,
        out_type=jax.ShapeDtypeStruct((total, HIDDEN), jnp.float32),
        mesh=mesh,
        scratch_types=(
            [pltpu.VMEM((b_per_w,), jnp.int32),
             pltpu.VMEM_SHARED((NUM_SUBCORES * NSLOT * CHUNK, HIDDEN),
                               jnp.float32)]
            + [pltpu.VMEM((CHUNK, HIDDEN), jnp.float32) for _ in range(NBUF)]
            + [pltpu.SemaphoreType.DMA for _ in range(2 * NBUF + NSLOT)]
        ),
    )(word_embeddings, ids)
    return out.reshape(batch, seq, HIDDEN)


# speedup vs baseline: 1.0882x; 1.0882x over previous
import functools

import jax
import jax.numpy as jnp
from jax import lax
from jax.experimental import pallas as pl
from jax.experimental.pallas import tpu as pltpu
from jax.experimental.pallas import tpu_sc as plsc

HIDDEN = 2048
NUM_CORES = 2
NUM_SUBCORES = 16
NUM_WORKERS = NUM_CORES * NUM_SUBCORES
CHUNK = 8
NBUF = 4


def _emb_body(table_hbm, ids_hbm, out_hbm, idx_v, *rest):
    bufs = rest[:NBUF]
    gsems = rest[NBUF:2 * NBUF]
    osems = rest[2 * NBUF:]
    b_per_w = idx_v.shape[0]
    nchunk = b_per_w // CHUNK
    ngroup = nchunk // NBUF
    wid = lax.axis_index("s") * NUM_CORES + lax.axis_index("c")
    base = wid * b_per_w
    pltpu.sync_copy(ids_hbm.at[pl.ds(base, b_per_w)], idx_v)

    def g_start(c, b):
        pltpu.async_copy(
            table_hbm.at[idx_v.at[pl.ds(c * CHUNK, CHUNK)]], bufs[b], gsems[b]
        )

    def g_wait(b):
        pltpu.make_async_copy(table_hbm.at[pl.ds(0, CHUNK)], bufs[b],
                              gsems[b]).wait()

    def o_start(c, b):
        pltpu.async_copy(bufs[b], out_hbm.at[pl.ds(base + c * CHUNK, CHUNK)],
                         osems[b])

    def o_wait(b):
        pltpu.make_async_copy(bufs[b], out_hbm.at[pl.ds(base, CHUNK)],
                              osems[b]).wait()

    def group_step(p, carry):
        for b in range(NBUF):

            @pl.when(p > 0)
            def _():
                o_wait(b)

            g_start(p * NBUF + b, b)
        for b in range(NBUF):
            g_wait(b)
            o_start(p * NBUF + b, b)
        return carry

    lax.fori_loop(0, ngroup, group_step, 0)
    for b in range(NBUF):
        o_wait(b)


def kernel(input_ids, word_embeddings):
    batch, seq = input_ids.shape
    total = batch * seq
    b_per_w = total // NUM_WORKERS
    ids = input_ids.reshape(total).astype(jnp.int32)

    mesh = plsc.VectorSubcoreMesh(core_axis_name="c", subcore_axis_name="s")
    out = pl.kernel(
        _emb_body,
        out_type=jax.ShapeDtypeStruct((total, HIDDEN), jnp.float32),
        mesh=mesh,
        scratch_types=(
            [pltpu.VMEM((b_per_w,), jnp.int32)]
            + [pltpu.VMEM((CHUNK, HIDDEN), jnp.float32) for _ in range(NBUF)]
            + [pltpu.SemaphoreType.DMA for _ in range(2 * NBUF)]
        ),
    )(word_embeddings, ids)
    return out.reshape(batch, seq, HIDDEN)
